# Initial kernel scaffold; baseline (speedup 1.0000x reference)
import jax
import jax.numpy as jnp
from jax.experimental import pallas as pl


def kernel(q, k, v, null_keys, null_values):
    raise NotImplementedError("write your pallas kernel here")


# speedup vs baseline: 7.1886x; 7.1886x over previous
import numpy as np
import jax
import jax.numpy as jnp
from jax.experimental import pallas as pl
from jax.experimental.pallas import tpu as pltpu

BSZ = 128
HIGHEST = jax.lax.Precision.HIGHEST


def _roll_left(x, n):
    return jnp.concatenate([x[n:], x[:n]], axis=0)


def _roll_right(x, n):
    return jnp.concatenate([x[-n:], x[:-n]], axis=0)


def _make_body(b, h, t, dh, a_consts):
    nb = t // BSZ
    hh = h // 2
    scale = np.float32(dh ** -0.5)

    def body(w_ref, q_ref, k_ref, v_ref, nk_ref, nv_ref,
             out_ref, kext_s, vext_s, outs_s, idx_s):
        i = pl.program_id(0)
        rot = (i % h) >= hh

        qw = jnp.where(rot, _roll_left(q_ref[0, 0], BSZ - 1), q_ref[0, 0])
        kw = jnp.where(rot, _roll_left(k_ref[0, 0], BSZ - 1), k_ref[0, 0])
        vw = jnp.where(rot, _roll_left(v_ref[0, 0], BSZ - 1), v_ref[0, 0])

        kext_s[0:BSZ, :] = jnp.broadcast_to(nk_ref[0], (BSZ, dh))
        kext_s[BSZ:, :] = kw
        vext_s[0:BSZ, :] = jnp.broadcast_to(nv_ref[0], (BSZ, dh))
        vext_s[BSZ:, :] = vw

        kww = kw * w_ref[:, :]
        acc_q = jnp.zeros((1, dh), jnp.float32)
        acc_k = jnp.zeros((1, dh), jnp.float32)
        sq_rows = []
        sk_rows = []
        for u in range(nb):
            qb = qw[u * BSZ:(u + 1) * BSZ]
            kb = kw[u * BSZ:(u + 1) * BSZ]
            sq_rows.append((acc_q + qb[0:1]) * np.float32(1.0 / (u * BSZ + 1)))
            kwsum = kww[u * BSZ:(u + 1) * BSZ].sum(axis=0, keepdims=True)
            sk_rows.append(acc_k * np.float32(a_consts[u]) + kwsum)
            acc_q = acc_q + qb.sum(axis=0, keepdims=True)
            acc_k = acc_k + kb.sum(axis=0, keepdims=True)

        SQ = jnp.concatenate(sq_rows, axis=0)
        SK = jnp.concatenate([jnp.zeros((1, dh), jnp.float32)] + sk_rows,
                             axis=0)
        R = jax.lax.dot_general(SQ, SK, (((1,), (1,)), ((), ())),
                                precision=HIGHEST,
                                preferred_element_type=jnp.float32)
        ir = jax.lax.broadcasted_iota(jnp.int32, (nb, nb + 1), 0)
        jc = jax.lax.broadcasted_iota(jnp.int32, (nb, nb + 1), 1)
        Rm = jnp.where(jc <= ir, R * scale, -jnp.finfo(jnp.float32).max)
        m = jnp.max(Rm, axis=1, keepdims=True)
        e = jnp.exp(Rm - m)
        p = e / jnp.sum(e, axis=1, keepdims=True)
        Rz = jnp.where(jc < ir, p, 0.0)
        mx = jnp.max(Rz, axis=1, keepdims=True)
        cand = jnp.where(Rz >= mx, jc.astype(jnp.float32), np.float32(1e9))
        idxf = jnp.min(cand, axis=1, keepdims=True)
        idx_s[...] = jnp.broadcast_to(
            idxf.astype(jnp.int32) * BSZ, (nb, BSZ))

        ii = jax.lax.broadcasted_iota(jnp.int32, (BSZ, BSZ), 0)
        jj = jax.lax.broadcasted_iota(jnp.int32, (BSZ, BSZ), 1)
        own_causal = jj <= ii

        group = []
        for u in range(nb):
            qu = qw[u * BSZ:(u + 1) * BSZ]
            ku = kw[u * BSZ:(u + 1) * BSZ]
            vu = vw[u * BSZ:(u + 1) * BSZ]
            off = pl.multiple_of(idx_s[u, 0], BSZ)
            kg = kext_s[pl.ds(off, BSZ), :]
            vg = vext_s[pl.ds(off, BSZ), :]
            val = mx[u:u + 1, 0:1]

            dots_own = jax.lax.dot_general(
                qu, ku, (((1,), (1,)), ((), ())),
                preferred_element_type=jnp.float32) * scale
            dots_g = jax.lax.dot_general(
                qu, kg, (((1,), (1,)), ((), ())),
                preferred_element_type=jnp.float32) * (scale * val)

            e_own = jnp.exp(dots_own)
            e_g = jnp.exp(dots_g)
            if u == nb - 1:
                own_mask = own_causal & ~(rot & (jj == 0) & (ii >= 1))
                g_mask = jnp.logical_or(~rot, ii == 0)
                e_g = jnp.where(g_mask, e_g, 0.0)
            else:
                own_mask = own_causal
            e_own = jnp.where(own_mask, e_own, 0.0)

            p_un = jnp.concatenate([e_g, e_own], axis=1)
            v2 = jnp.concatenate([vg * val, vu], axis=0)
            acc = jax.lax.dot_general(
                p_un, v2, (((1,), (0,)), ((), ())),
                preferred_element_type=jnp.float32)
            s = jnp.sum(p_un, axis=1, keepdims=True)
            group.append(acc / s)
            gsz = min(8, nb)
            if len(group) == gsz:
                base = (u - gsz + 1) * BSZ
                outs_s[base:base + gsz * BSZ, :] = jnp.concatenate(group,
                                                                   axis=0)
                group = []

        ow = outs_s[:, :]
        out_ref[0, 0] = jnp.where(rot, _roll_right(ow, BSZ - 1), ow)

    return body


def kernel(q, k, v, null_keys, null_values):
    b, h, t, dh = q.shape
    nb = t // BSZ
    bh = b * h

    pos = np.arange(t, dtype=np.float64).reshape(nb, BSZ)
    wmat = 1.0 / (pos + 1.0)
    wsuf = np.cumsum(wmat[:, ::-1], axis=1)[:, ::-1]
    wfull = jnp.asarray(
        np.broadcast_to(wsuf.reshape(t, 1), (t, dh)), dtype=jnp.float32)
    a_consts = [float(wsuf[u, 0]) for u in range(nb)]

    fused = pl.pallas_call(
        _make_body(b, h, t, dh, a_consts),
        grid=(bh,),
        in_specs=[
            pl.BlockSpec((t, dh), lambda i: (0, 0)),
            pl.BlockSpec((1, 1, t, dh), lambda i: (i // h, i % h, 0, 0)),
            pl.BlockSpec((1, 1, t, dh), lambda i: (i // h, i % h, 0, 0)),
            pl.BlockSpec((1, 1, t, dh), lambda i: (i // h, i % h, 0, 0)),
            pl.BlockSpec((1, 1, dh), lambda i: (i % h, 0, 0)),
            pl.BlockSpec((1, 1, dh), lambda i: (i % h, 0, 0)),
        ],
        out_specs=pl.BlockSpec((1, 1, t, dh), lambda i: (i // h, i % h, 0, 0)),
        out_shape=jax.ShapeDtypeStruct((b, h, t, dh), jnp.float32),
        scratch_shapes=[
            pltpu.VMEM(((nb + 1) * BSZ, dh), jnp.float32),
            pltpu.VMEM(((nb + 1) * BSZ, dh), jnp.float32),
            pltpu.VMEM((t, dh), jnp.float32),
            pltpu.VMEM((nb, BSZ), jnp.int32),
        ],
        compiler_params=pltpu.CompilerParams(
            dimension_semantics=("arbitrary",)),
    )
    return fused(wfull, q, k, v, null_keys, null_values)


# speedup vs baseline: 7.7937x; 1.0842x over previous
import numpy as np
import jax
import jax.numpy as jnp
from jax.experimental import pallas as pl
from jax.experimental.pallas import tpu as pltpu

BSZ = 128
HIGHEST = jax.lax.Precision.HIGHEST


def _make_body(b, h, t, dh):
    nb = t // BSZ
    nph = h // 2
    scale = np.float32(dh ** -0.5)
    r = BSZ - 1
    gsz = min(8, nb)

    def body(w_ref, a_ref, q_ref, k_ref, v_ref, nk_ref, nv_ref,
             out_ref, idx_s, wrap_s):
        i = pl.program_id(0)
        rot = (i % nph) >= (nph // 2)
        roff = jnp.where(rot, r, 0)

        for sub in range(2):
            wb = sub * 3 * BSZ
            for j, ref in enumerate((q_ref, k_ref, v_ref)):
                wrap_s[wb + j * BSZ:wb + j * BSZ + 1, :] = \
                    ref[0, sub, t - 1:t, :]
                wrap_s[wb + j * BSZ + 1:wb + (j + 1) * BSZ, :] = \
                    ref[0, sub, 0:r, :]

        def bucket(ref, sub, u, j):
            if u < nb - 1:
                return ref[0, sub, pl.ds(u * BSZ + roff, BSZ), :]
            aligned = ref[0, sub, (nb - 1) * BSZ:t, :]
            wb = sub * 3 * BSZ + j * BSZ
            return jnp.where(rot, wrap_s[wb:wb + BSZ, :], aligned)

        def prefix_excl(x):
            p = jnp.concatenate([jnp.zeros((1, dh), jnp.float32), x[:-1]],
                                axis=0)
            s = 1
            while s < nb:
                p = p + jnp.concatenate(
                    [jnp.zeros((s, dh), jnp.float32), p[:-s]], axis=0)
                s *= 2
            return p

        posn = (jax.lax.broadcasted_iota(jnp.int32, (nb, dh), 0) * BSZ
                + 1).astype(jnp.float32)
        ir = jax.lax.broadcasted_iota(jnp.int32, (nb, nb + 1), 0)
        jc = jax.lax.broadcasted_iota(jnp.int32, (nb, nb + 1), 1)

        mx_subs, idx_rows = [], []
        for sub in range(2):
            qsum_rows, ksum_rows, kwsum_rows, qfirst_rows = [], [], [], []
            for u in range(nb):
                qb = bucket(q_ref, sub, u, 0)
                kb = bucket(k_ref, sub, u, 1)
                qsum_rows.append(qb.sum(axis=0, keepdims=True))
                ksum_rows.append(kb.sum(axis=0, keepdims=True))
                kwsum_rows.append(
                    (kb * w_ref[u * BSZ:(u + 1) * BSZ, :]).sum(
                        axis=0, keepdims=True))
                qfirst_rows.append(qb[0:1])

            qsum = jnp.concatenate(qsum_rows, axis=0)
            ksum = jnp.concatenate(ksum_rows, axis=0)
            kwsum = jnp.concatenate(kwsum_rows, axis=0)
            qfirst = jnp.concatenate(qfirst_rows, axis=0)

            SQ = (prefix_excl(qsum) + qfirst) / posn
            sk = prefix_excl(ksum) * a_ref[:, :] + kwsum
            SK = jnp.concatenate([jnp.zeros((1, dh), jnp.float32), sk],
                                 axis=0)
            R = jax.lax.dot_general(SQ, SK, (((1,), (1,)), ((), ())),
                                    precision=HIGHEST,
                                    preferred_element_type=jnp.float32)
            Rm = jnp.where(jc <= ir, R * scale,
                           -jnp.finfo(jnp.float32).max)
            m = jnp.max(Rm, axis=1, keepdims=True)
            e = jnp.exp(Rm - m)
            p = e / jnp.sum(e, axis=1, keepdims=True)
            Rz = jnp.where(jc < ir, p, 0.0)
            mx = jnp.max(Rz, axis=1, keepdims=True)
            cand = jnp.where(Rz >= mx, jc.astype(jnp.float32),
                             np.float32(1e9))
            idxf = jnp.min(cand, axis=1, keepdims=True)
            mx_subs.append(mx)
            idx_rows.append(jnp.broadcast_to(idxf.astype(jnp.int32),
                                             (nb, BSZ)))
        idx_s[...] = jnp.concatenate(idx_rows, axis=0)

        nulls = [
            (jnp.broadcast_to(nk_ref[sub, 0:1], (BSZ, dh)),
             jnp.broadcast_to(nv_ref[sub, 0:1], (BSZ, dh)))
            for sub in range(2)
        ]

        ii = jax.lax.broadcasted_iota(jnp.int32, (BSZ, BSZ), 0)
        jj = jax.lax.broadcasted_iota(jnp.int32, (BSZ, BSZ), 1)
        own_causal = jj <= ii

        def flush(groups, u):
            base = (u - gsz + 1) * BSZ
            ow = jnp.stack([jnp.concatenate(g, axis=0) for g in groups],
                           axis=0)
            last = base + gsz * BSZ == t

            @pl.when(rot)
            def _():
                if last:
                    out_ref[0, :, base + r:t, :] = ow[:, 0:gsz * BSZ - r, :]
                    out_ref[0, :, 0:r, :] = ow[:, gsz * BSZ - r:, :]
                else:
                    out_ref[0, :, base + r:base + gsz * BSZ + r, :] = ow

            @pl.when(jnp.logical_not(rot))
            def _():
                out_ref[0, :, base:base + gsz * BSZ, :] = ow

        groups = [[], []]
        for u in range(nb):
            for sub in range(2):
                qu = bucket(q_ref, sub, u, 0)
                ku = bucket(k_ref, sub, u, 1)
                vu = bucket(v_ref, sub, u, 2)
                qu16 = (qu * scale).astype(jnp.bfloat16)

                dots_own = jax.lax.dot_general(
                    qu16, ku.astype(jnp.bfloat16), (((1,), (1,)), ((), ())),
                    preferred_element_type=jnp.float32)
                e_own = jnp.exp(dots_own)
                if u == nb - 1:
                    own_mask = own_causal & ~(rot & (jj == 0) & (ii >= 1))
                else:
                    own_mask = own_causal
                e_own = jnp.where(own_mask, e_own, 0.0)

                if u == 0:
                    acc = jax.lax.dot_general(
                        e_own.astype(jnp.bfloat16), vu.astype(jnp.bfloat16),
                        (((1,), (0,)), ((), ())),
                        preferred_element_type=jnp.float32)
                    s = jnp.sum(e_own, axis=1, keepdims=True) \
                        + np.float32(BSZ)
                    groups[sub].append(acc / s)
                    continue

                iu = idx_s[sub * nb + u, 0]
                offg = jnp.maximum(iu - 1, 0) * BSZ + roff
                kgl = k_ref[0, sub, pl.ds(offg, BSZ), :]
                vgl = v_ref[0, sub, pl.ds(offg, BSZ), :]
                isnull = iu == 0
                kg = jnp.where(isnull, nulls[sub][0], kgl)
                vg = jnp.where(isnull, nulls[sub][1], vgl)
                val = mx_subs[sub][u:u + 1, 0:1]

                dots_g = jax.lax.dot_general(
                    qu16, (kg * val).astype(jnp.bfloat16),
                    (((1,), (1,)), ((), ())),
                    preferred_element_type=jnp.float32)
                e_g = jnp.exp(dots_g)
                if u == nb - 1:
                    g_mask = jnp.logical_or(~rot, ii == 0)
                    e_g = jnp.where(g_mask, e_g, 0.0)

                p_un = jnp.concatenate([e_g, e_own], axis=1)
                v2 = jnp.concatenate([vg * val, vu], axis=0)
                acc = jax.lax.dot_general(
                    p_un.astype(jnp.bfloat16), v2.astype(jnp.bfloat16),
                    (((1,), (0,)), ((), ())),
                    preferred_element_type=jnp.float32)
                s = jnp.sum(p_un, axis=1, keepdims=True)
                groups[sub].append(acc / s)
            if len(groups[0]) == gsz:
                flush(groups, u)
                groups = [[], []]

    return body


def kernel(q, k, v, null_keys, null_values):
    b, h, t, dh = q.shape
    nb = t // BSZ
    nph = h // 2

    pos = np.arange(t, dtype=np.float64).reshape(nb, BSZ)
    wmat = 1.0 / (pos + 1.0)
    wsuf = np.cumsum(wmat[:, ::-1], axis=1)[:, ::-1]
    wfull = jnp.asarray(
        np.broadcast_to(wsuf.reshape(t, 1), (t, dh)), dtype=jnp.float32)
    afull = jnp.asarray(
        np.broadcast_to(wsuf[:, 0:1], (nb, dh)), dtype=jnp.float32)

    fused = pl.pallas_call(
        _make_body(b, h, t, dh),
        grid=(b * nph,),
        in_specs=[
            pl.BlockSpec((t, dh), lambda i: (0, 0)),
            pl.BlockSpec((nb, dh), lambda i: (0, 0)),
            pl.BlockSpec((1, 2, t, dh), lambda i: (i // nph, i % nph, 0, 0)),
            pl.BlockSpec((1, 2, t, dh), lambda i: (i // nph, i % nph, 0, 0)),
            pl.BlockSpec((1, 2, t, dh), lambda i: (i // nph, i % nph, 0, 0)),
            pl.BlockSpec((2, 1, dh), lambda i: (i % nph, 0, 0)),
            pl.BlockSpec((2, 1, dh), lambda i: (i % nph, 0, 0)),
        ],
        out_specs=pl.BlockSpec((1, 2, t, dh),
                               lambda i: (i // nph, i % nph, 0, 0)),
        out_shape=jax.ShapeDtypeStruct((b, h, t, dh), jnp.float32),
        scratch_shapes=[
            pltpu.VMEM((2 * nb, BSZ), jnp.int32),
            pltpu.VMEM((6 * BSZ, dh), jnp.float32),
        ],
        compiler_params=pltpu.CompilerParams(
            dimension_semantics=("arbitrary",)),
    )
    return fused(wfull, afull, q, k, v, null_keys, null_values)


# speedup vs baseline: 8.0006x; 1.0265x over previous
import numpy as np
import jax
import jax.numpy as jnp
from jax.experimental import pallas as pl
from jax.experimental.pallas import tpu as pltpu

BSZ = 128
HIGHEST = jax.lax.Precision.HIGHEST


def _make_body(b, h, t, dh, is_rot, has_prev):
    nb = t // BSZ
    scale = np.float32(dh ** -0.5)
    qscale = np.float32(dh ** -0.5 * np.log2(np.e))
    r = BSZ - 1
    gsz = min(8, nb)

    def body(*refs):
        if has_prev:
            w_ref, a_ref, q_ref, k_ref, v_ref, nk_ref, nv_ref, _prev, \
                out_ref, idx_s, *stage = refs
        else:
            w_ref, a_ref, q_ref, k_ref, v_ref, nk_ref, nv_ref, \
                out_ref, idx_s, *stage = refs

        if is_rot:
            for sub in range(2):
                for src, dst in zip((q_ref, k_ref, v_ref), stage):
                    dst[sub * t:sub * t + t - r, :] = src[0, sub, r:t, :]
                    dst[sub * t + t - r:(sub + 1) * t, :] = \
                        src[0, sub, 0:r, :]

            def bucket(which, sub, u):
                ref = stage[which]
                return ref[sub * t + u * BSZ:sub * t + (u + 1) * BSZ, :]

            def gather(which, sub, offg):
                ref = stage[which]
                return ref[pl.ds(sub * t + offg, BSZ), :]
        else:
            def bucket(which, sub, u):
                ref = (q_ref, k_ref, v_ref)[which]
                return ref[0, sub, u * BSZ:(u + 1) * BSZ, :]

            def gather(which, sub, offg):
                ref = (q_ref, k_ref, v_ref)[which]
                return ref[0, sub, pl.ds(offg, BSZ), :]

        def prefix_excl(x):
            p = jnp.concatenate([jnp.zeros((1, dh), jnp.float32), x[:-1]],
                                axis=0)
            s = 1
            while s < nb:
                p = p + jnp.concatenate(
                    [jnp.zeros((s, dh), jnp.float32), p[:-s]], axis=0)
                s *= 2
            return p

        posn = (jax.lax.broadcasted_iota(jnp.int32, (nb, dh), 0) * BSZ
                + 1).astype(jnp.float32)
        ir = jax.lax.broadcasted_iota(jnp.int32, (nb, nb + 1), 0)
        jc = jax.lax.broadcasted_iota(jnp.int32, (nb, nb + 1), 1)

        mx_subs, idx_rows = [], []
        for sub in range(2):
            qsum_rows, ksum_rows, kwsum_rows, qfirst_rows = [], [], [], []
            for u in range(nb):
                qb = bucket(0, sub, u)
                kb = bucket(1, sub, u)
                qsum_rows.append(qb.sum(axis=0, keepdims=True))
                ksum_rows.append(kb.sum(axis=0, keepdims=True))
                kwsum_rows.append(
                    (kb * w_ref[u * BSZ:(u + 1) * BSZ, :]).sum(
                        axis=0, keepdims=True))
                qfirst_rows.append(qb[0:1])

            qsum = jnp.concatenate(qsum_rows, axis=0)
            ksum = jnp.concatenate(ksum_rows, axis=0)
            kwsum = jnp.concatenate(kwsum_rows, axis=0)
            qfirst = jnp.concatenate(qfirst_rows, axis=0)

            SQ = (prefix_excl(qsum) + qfirst) / posn
            sk = prefix_excl(ksum) * a_ref[:, :] + kwsum
            SK = jnp.concatenate([jnp.zeros((1, dh), jnp.float32), sk],
                                 axis=0)
            R = jax.lax.dot_general(SQ, SK, (((1,), (1,)), ((), ())),
                                    precision=HIGHEST,
                                    preferred_element_type=jnp.float32)
            Rm = jnp.where(jc <= ir, R * scale,
                           -jnp.finfo(jnp.float32).max)
            m = jnp.max(Rm, axis=1, keepdims=True)
            e = jnp.exp(Rm - m)
            p = e / jnp.sum(e, axis=1, keepdims=True)
            Rz = jnp.where(jc < ir, p, 0.0)
            mx = jnp.max(Rz, axis=1, keepdims=True)
            cand = jnp.where(Rz >= mx, jc.astype(jnp.float32),
                             np.float32(1e9))
            idxf = jnp.min(cand, axis=1, keepdims=True)
            mx_subs.append(mx)
            idx_rows.append(jnp.broadcast_to(idxf.astype(jnp.int32),
                                             (nb, BSZ)))
        idx_s[...] = jnp.concatenate(idx_rows, axis=0)

        nulls = [
            (jnp.broadcast_to(nk_ref[sub, 0:1], (BSZ, dh)),
             jnp.broadcast_to(nv_ref[sub, 0:1], (BSZ, dh)))
            for sub in range(2)
        ]

        ii = jax.lax.broadcasted_iota(jnp.int32, (BSZ, BSZ), 0)
        jj = jax.lax.broadcasted_iota(jnp.int32, (BSZ, BSZ), 1)
        own_causal = jj <= ii

        def flush(groups, u):
            base = (u - gsz + 1) * BSZ
            ow = jnp.stack([jnp.concatenate(g, axis=0) for g in groups],
                           axis=0)
            if not is_rot:
                out_ref[0, :, base:base + gsz * BSZ, :] = ow
            elif base + gsz * BSZ == t:
                out_ref[0, :, base + r:t, :] = ow[:, 0:gsz * BSZ - r, :]
                out_ref[0, :, 0:r, :] = ow[:, gsz * BSZ - r:, :]
            else:
                out_ref[0, :, base + r:base + gsz * BSZ + r, :] = ow

        groups = [[], []]
        for u in range(nb):
            for sub in range(2):
                qu = bucket(0, sub, u)
                ku = bucket(1, sub, u)
                vu = bucket(2, sub, u)
                qu16 = (qu * qscale).astype(jnp.bfloat16)
                vu16 = vu.astype(jnp.bfloat16)

                dots_own = jax.lax.dot_general(
                    qu16, ku.astype(jnp.bfloat16), (((1,), (1,)), ((), ())),
                    preferred_element_type=jnp.float32)
                e_own = jnp.exp2(dots_own)
                if u == nb - 1 and is_rot:
                    own_mask = own_causal & ~((jj == 0) & (ii >= 1))
                else:
                    own_mask = own_causal
                e_own = jnp.where(own_mask, e_own, 0.0)

                if u == 0:
                    acc = jax.lax.dot_general(
                        e_own.astype(jnp.bfloat16), vu16,
                        (((1,), (0,)), ((), ())),
                        preferred_element_type=jnp.float32)
                    s = jnp.sum(e_own, axis=1, keepdims=True) \
                        + np.float32(BSZ)
                    groups[sub].append(acc / s)
                    continue

                iu = idx_s[sub * nb + u, 0]
                offg = pl.multiple_of(jnp.maximum(iu - 1, 0) * BSZ, BSZ)
                kgl = gather(1, sub, offg)
                vgl = gather(2, sub, offg)
                isnull = iu == 0
                kg = jnp.where(isnull, nulls[sub][0], kgl)
                vg = jnp.where(isnull, nulls[sub][1], vgl)
                val = mx_subs[sub][u:u + 1, 0:1]

                dots_g = jax.lax.dot_general(
                    qu16, (kg * val).astype(jnp.bfloat16),
                    (((1,), (1,)), ((), ())),
                    preferred_element_type=jnp.float32)
                e_g = jnp.exp2(dots_g)
                if u == nb - 1 and is_rot:
                    e_g = jnp.where(ii == 0, e_g, 0.0)

                acc = jax.lax.dot_general(
                    e_g.astype(jnp.bfloat16),
                    (vg * val).astype(jnp.bfloat16),
                    (((1,), (0,)), ((), ())),
                    preferred_element_type=jnp.float32) + jax.lax.dot_general(
                    e_own.astype(jnp.bfloat16), vu16,
                    (((1,), (0,)), ((), ())),
                    preferred_element_type=jnp.float32)
                s = (jnp.sum(e_g, axis=1, keepdims=True)
                     + jnp.sum(e_own, axis=1, keepdims=True))
                groups[sub].append(acc / s)
            if len(groups[0]) == gsz:
                flush(groups, u)
                groups = [[], []]

    return body


def kernel(q, k, v, null_keys, null_values):
    b, h, t, dh = q.shape
    nb = t // BSZ
    nph = h // 2
    nhalf = nph // 2

    pos = np.arange(t, dtype=np.float64).reshape(nb, BSZ)
    wmat = 1.0 / (pos + 1.0)
    wsuf = np.cumsum(wmat[:, ::-1], axis=1)[:, ::-1]
    wfull = jnp.asarray(
        np.broadcast_to(wsuf.reshape(t, 1), (t, dh)), dtype=jnp.float32)
    afull = jnp.asarray(
        np.broadcast_to(wsuf[:, 0:1], (nb, dh)), dtype=jnp.float32)

    def make_call(is_rot, has_prev):
        pair0 = nhalf if is_rot else 0

        def rowmap(i, base=pair0):
            return (i // nhalf, base + i % nhalf, 0, 0)

        def nullmap(i, base=pair0):
            return (base + i % nhalf, 0, 0)

        in_specs = [
            pl.BlockSpec((t, dh), lambda i: (0, 0)),
            pl.BlockSpec((nb, dh), lambda i: (0, 0)),
            pl.BlockSpec((1, 2, t, dh), rowmap),
            pl.BlockSpec((1, 2, t, dh), rowmap),
            pl.BlockSpec((1, 2, t, dh), rowmap),
            pl.BlockSpec((2, 1, dh), nullmap),
            pl.BlockSpec((2, 1, dh), nullmap),
        ]
        aliases = {}
        if has_prev:
            in_specs.append(pl.BlockSpec(memory_space=pltpu.MemorySpace.HBM))
            aliases = {7: 0}
        scratch = [pltpu.VMEM((2 * nb, BSZ), jnp.int32)]
        if is_rot:
            scratch += [pltpu.VMEM((2 * t, dh), jnp.float32)] * 3
        return pl.pallas_call(
            _make_body(b, h, t, dh, is_rot, has_prev),
            grid=(b * nhalf,),
            in_specs=in_specs,
            out_specs=pl.BlockSpec((1, 2, t, dh), rowmap),
            out_shape=jax.ShapeDtypeStruct((b, h, t, dh), jnp.float32),
            scratch_shapes=scratch,
            input_output_aliases=aliases,
            compiler_params=pltpu.CompilerParams(
                dimension_semantics=("arbitrary",)),
        )

    out1 = make_call(False, False)(wfull, afull, q, k, v,
                                   null_keys, null_values)
    return make_call(True, True)(wfull, afull, q, k, v,
                                 null_keys, null_values, out1)


# speedup vs baseline: 9.8049x; 1.2255x over previous
import numpy as np
import jax
import jax.numpy as jnp
from jax.experimental import pallas as pl
from jax.experimental.pallas import tpu as pltpu

BSZ = 128
HIGHEST = jax.lax.Precision.HIGHEST


def _make_body(b, h, t, dh):
    nb = t // BSZ
    scale = np.float32(dh ** -0.5)
    qscale = np.float32(dh ** -0.5 * np.log2(np.e))
    r = BSZ - 1
    gsz = min(8, nb)

    def body(w_ref, a_ref, q_ref, k_ref, v_ref, nk_ref, nv_ref,
             out_ref, idx_s, qst, kst, vst):
        stage = (qst, kst, vst)
        for sub in range(1):
            for src, dst in zip((q_ref, k_ref, v_ref), stage):
                dst[sub * t:sub * t + t - r, :] = src[0, 1, sub, r:t, :]
                dst[sub * t + t - r:(sub + 1) * t, :] = src[0, 1, sub, 0:r, :]

        def bucket(which, half, sub, u):
            if half:
                ref = stage[which]
                return ref[sub * t + u * BSZ:sub * t + (u + 1) * BSZ, :]
            ref = (q_ref, k_ref, v_ref)[which]
            return ref[0, 0, sub, u * BSZ:(u + 1) * BSZ, :]

        def gather(which, half, sub, offg):
            if half:
                return stage[which][pl.ds(sub * t + offg, BSZ), :]
            ref = (q_ref, k_ref, v_ref)[which]
            return ref[0, 0, sub, pl.ds(offg, BSZ), :]

        def prefix_excl(x):
            p = jnp.concatenate([jnp.zeros((1, dh), jnp.float32), x[:-1]],
                                axis=0)
            s = 1
            while s < nb:
                p = p + jnp.concatenate(
                    [jnp.zeros((s, dh), jnp.float32), p[:-s]], axis=0)
                s *= 2
            return p

        posn = (jax.lax.broadcasted_iota(jnp.int32, (nb, dh), 0) * BSZ
                + 1).astype(jnp.float32)
        ir = jax.lax.broadcasted_iota(jnp.int32, (nb, nb + 1), 0)
        jc = jax.lax.broadcasted_iota(jnp.int32, (nb, nb + 1), 1)

        mx_all, idx_rows = {}, []
        for half in range(2):
            for sub in range(1):
                qsums, ksums, kwsums, qfirsts = [], [], [], []
                for u in range(nb):
                    qb = bucket(0, half, sub, u)
                    kb = bucket(1, half, sub, u)
                    qsums.append(qb.sum(axis=0, keepdims=True))
                    ksums.append(kb.sum(axis=0, keepdims=True))
                    kwsums.append(
                        (kb * w_ref[u * BSZ:(u + 1) * BSZ, :]).sum(
                            axis=0, keepdims=True))
                    qfirsts.append(qb[0:1])

                qsum = jnp.concatenate(qsums, axis=0)
                ksum = jnp.concatenate(ksums, axis=0)
                kwsum = jnp.concatenate(kwsums, axis=0)
                qfirst = jnp.concatenate(qfirsts, axis=0)

                SQ = (prefix_excl(qsum) + qfirst) / posn
                sk = prefix_excl(ksum) * a_ref[:, :] + kwsum
                SK = jnp.concatenate(
                    [jnp.zeros((1, dh), jnp.float32), sk], axis=0)
                R = jax.lax.dot_general(SQ, SK, (((1,), (1,)), ((), ())),
                                        precision=HIGHEST,
                                        preferred_element_type=jnp.float32)
                Rm = jnp.where(jc <= ir, R * scale,
                               -jnp.finfo(jnp.float32).max)
                m = jnp.max(Rm, axis=1, keepdims=True)
                e = jnp.exp(Rm - m)
                p = e / jnp.sum(e, axis=1, keepdims=True)
                Rz = jnp.where(jc < ir, p, 0.0)
                mx = jnp.max(Rz, axis=1, keepdims=True)
                cand = jnp.where(Rz >= mx, jc.astype(jnp.float32),
                                 np.float32(1e9))
                idxf = jnp.min(cand, axis=1, keepdims=True)
                mx_all[(half, sub)] = mx
                idx_rows.append(jnp.broadcast_to(idxf.astype(jnp.int32),
                                                 (nb, BSZ)))
        idx_s[...] = jnp.concatenate(idx_rows, axis=0)

        nulls = {
            (half, sub): (
                jnp.broadcast_to(nk_ref[half, sub], (BSZ, dh)),
                jnp.broadcast_to(nv_ref[half, sub], (BSZ, dh)))
            for half in range(2) for sub in range(1)
        }

        ii = jax.lax.broadcasted_iota(jnp.int32, (BSZ, BSZ), 0)
        jj = jax.lax.broadcasted_iota(jnp.int32, (BSZ, BSZ), 1)
        own_causal = jj <= ii

        def flush(groups, u):
            base = (u - gsz + 1) * BSZ
            ow0 = jnp.stack([jnp.concatenate(groups[(0, s)], axis=0)
                             for s in range(1)], axis=0)
            ow1 = jnp.stack([jnp.concatenate(groups[(1, s)], axis=0)
                             for s in range(1)], axis=0)
            out_ref[0, 0, :, base:base + gsz * BSZ, :] = ow0
            if base + gsz * BSZ == t:
                out_ref[0, 1, :, base + r:t, :] = ow1[:, 0:gsz * BSZ - r, :]
                out_ref[0, 1, :, 0:r, :] = ow1[:, gsz * BSZ - r:, :]
            else:
                out_ref[0, 1, :, base + r:base + gsz * BSZ + r, :] = ow1

        groups = {(hf, s): [] for hf in range(2) for s in range(1)}
        for u in range(nb):
            for half in range(2):
                for sub in range(1):
                    qu = bucket(0, half, sub, u)
                    ku = bucket(1, half, sub, u)
                    vu = bucket(2, half, sub, u)
                    qu16 = (qu * qscale).astype(jnp.bfloat16)
                    vu16 = vu.astype(jnp.bfloat16)

                    dots_own = jax.lax.dot_general(
                        qu16, ku.astype(jnp.bfloat16),
                        (((1,), (1,)), ((), ())),
                        preferred_element_type=jnp.float32)
                    e_own = jnp.exp2(dots_own)
                    if u == nb - 1 and half:
                        own_mask = own_causal & ~((jj == 0) & (ii >= 1))
                    else:
                        own_mask = own_causal
                    e_own = jnp.where(own_mask, e_own, 0.0)

                    if u == 0:
                        acc = jax.lax.dot_general(
                            e_own.astype(jnp.bfloat16), vu16,
                            (((1,), (0,)), ((), ())),
                            preferred_element_type=jnp.float32)
                        s = jnp.sum(e_own, axis=1, keepdims=True) \
                            + np.float32(BSZ)
                        groups[(half, sub)].append(acc / s)
                        continue

                    iu = idx_s[(half + sub) * nb + u, 0]
                    offg = pl.multiple_of(
                        jnp.maximum(iu - 1, 0) * BSZ, BSZ)
                    kgl = gather(1, half, sub, offg)
                    vgl = gather(2, half, sub, offg)
                    isnull = iu == 0
                    kg = jnp.where(isnull, nulls[(half, sub)][0], kgl)
                    vg = jnp.where(isnull, nulls[(half, sub)][1], vgl)
                    val = mx_all[(half, sub)][u:u + 1, 0:1]

                    dots_g = jax.lax.dot_general(
                        qu16, (kg * val).astype(jnp.bfloat16),
                        (((1,), (1,)), ((), ())),
                        preferred_element_type=jnp.float32)
                    e_g = jnp.exp2(dots_g)
                    if u == nb - 1 and half:
                        e_g = jnp.where(ii == 0, e_g, 0.0)

                    acc = jax.lax.dot_general(
                        e_g.astype(jnp.bfloat16),
                        (vg * val).astype(jnp.bfloat16),
                        (((1,), (0,)), ((), ())),
                        preferred_element_type=jnp.float32) \
                        + jax.lax.dot_general(
                            e_own.astype(jnp.bfloat16), vu16,
                            (((1,), (0,)), ((), ())),
                            preferred_element_type=jnp.float32)
                    s = (jnp.sum(e_g, axis=1, keepdims=True)
                         + jnp.sum(e_own, axis=1, keepdims=True))
                    groups[(half, sub)].append(acc / s)
            if len(groups[(0, 0)]) == gsz:
                flush(groups, u)
                groups = {(hf, s): [] for hf in range(2) for s in range(1)}

    return body


def kernel(q, k, v, null_keys, null_values):
    b, h, t, dh = q.shape
    nb = t // BSZ
    hh = h // 2
    nhalf = hh

    pos = np.arange(t, dtype=np.float64).reshape(nb, BSZ)
    wmat = 1.0 / (pos + 1.0)
    wsuf = np.cumsum(wmat[:, ::-1], axis=1)[:, ::-1]
    wfull = jnp.asarray(
        np.broadcast_to(wsuf.reshape(t, 1), (t, dh)), dtype=jnp.float32)
    afull = jnp.asarray(
        np.broadcast_to(wsuf[:, 0:1], (nb, dh)), dtype=jnp.float32)

    q5 = q.reshape(b, 2, hh, t, dh)
    k5 = k.reshape(b, 2, hh, t, dh)
    v5 = v.reshape(b, 2, hh, t, dh)
    nk4 = null_keys.reshape(2, hh, 1, dh)
    nv4 = null_values.reshape(2, hh, 1, dh)

    def rowmap(i):
        return (i // nhalf, 0, i % nhalf, 0, 0)

    fused = pl.pallas_call(
        _make_body(b, h, t, dh),
        grid=(b * nhalf,),
        in_specs=[
            pl.BlockSpec((t, dh), lambda i: (0, 0)),
            pl.BlockSpec((nb, dh), lambda i: (0, 0)),
            pl.BlockSpec((1, 2, 1, t, dh), rowmap),
            pl.BlockSpec((1, 2, 1, t, dh), rowmap),
            pl.BlockSpec((1, 2, 1, t, dh), rowmap),
            pl.BlockSpec((2, 1, 1, dh), lambda i: (0, i % nhalf, 0, 0)),
            pl.BlockSpec((2, 1, 1, dh), lambda i: (0, i % nhalf, 0, 0)),
        ],
        out_specs=pl.BlockSpec((1, 2, 1, t, dh), rowmap),
        out_shape=jax.ShapeDtypeStruct((b, 2, hh, t, dh), jnp.float32),
        scratch_shapes=[
            pltpu.VMEM((2 * nb, BSZ), jnp.int32),
            pltpu.VMEM((t, dh), jnp.float32),
            pltpu.VMEM((t, dh), jnp.float32),
            pltpu.VMEM((t, dh), jnp.float32),
        ],
        compiler_params=pltpu.CompilerParams(
            dimension_semantics=("arbitrary",)),
    )
    out5 = fused(wfull, afull, q5, k5, v5, nk4, nv4)
    return out5.reshape(b, h, t, dh)


# speedup vs baseline: 9.8977x; 1.0095x over previous
import numpy as np
import jax
import jax.numpy as jnp
from jax.experimental import pallas as pl
from jax.experimental.pallas import tpu as pltpu

BSZ = 128
HIGHEST = jax.lax.Precision.HIGHEST


def _make_body(b, h, t, dh):
    nb = t // BSZ
    scale = np.float32(dh ** -0.5)
    qscale = np.float32(dh ** -0.5 * np.log2(np.e))
    r = BSZ - 1
    gsz = min(8, nb)

    def body(w_ref, a_ref, q_ref, k_ref, v_ref, nk_ref, nv_ref,
             out_ref, idx_s, qst, kst, vst):
        stage = (qst, kst, vst)
        for sub in range(1):
            for src, dst in zip((q_ref, k_ref, v_ref), stage):
                dst[sub * t:sub * t + t - r, :] = src[0, 1, sub, r:t, :]
                dst[sub * t + t - r:(sub + 1) * t, :] = src[0, 1, sub, 0:r, :]

        def bucket(which, half, sub, u):
            if half:
                ref = stage[which]
                return ref[sub * t + u * BSZ:sub * t + (u + 1) * BSZ, :]
            ref = (q_ref, k_ref, v_ref)[which]
            return ref[0, 0, sub, u * BSZ:(u + 1) * BSZ, :]

        def gather(which, half, sub, offg):
            if half:
                return stage[which][pl.ds(sub * t + offg, BSZ), :]
            ref = (q_ref, k_ref, v_ref)[which]
            return ref[0, 0, sub, pl.ds(offg, BSZ), :]

        def prefix_excl(x):
            p = jnp.concatenate([jnp.zeros((1, dh), jnp.float32), x[:-1]],
                                axis=0)
            s = 1
            while s < nb:
                p = p + jnp.concatenate(
                    [jnp.zeros((s, dh), jnp.float32), p[:-s]], axis=0)
                s *= 2
            return p

        posn = (jax.lax.broadcasted_iota(jnp.int32, (nb, dh), 0) * BSZ
                + 1).astype(jnp.float32)
        ir = jax.lax.broadcasted_iota(jnp.int32, (nb, nb + 1), 0)
        jc = jax.lax.broadcasted_iota(jnp.int32, (nb, nb + 1), 1)

        mx_all, idx_rows = {}, []
        for half in range(2):
            for sub in range(1):
                qsums, ksums, kwsums, qfirsts = [], [], [], []
                for u in range(nb):
                    qb = bucket(0, half, sub, u)
                    kb = bucket(1, half, sub, u)
                    qsums.append(qb.sum(axis=0, keepdims=True))
                    ksums.append(kb.sum(axis=0, keepdims=True))
                    kwsums.append(
                        (kb * w_ref[u * BSZ:(u + 1) * BSZ, :]).sum(
                            axis=0, keepdims=True))
                    qfirsts.append(qb[0:1])

                qsum = jnp.concatenate(qsums, axis=0)
                ksum = jnp.concatenate(ksums, axis=0)
                kwsum = jnp.concatenate(kwsums, axis=0)
                qfirst = jnp.concatenate(qfirsts, axis=0)

                SQ = (prefix_excl(qsum) + qfirst) / posn
                sk = prefix_excl(ksum) * a_ref[:, :] + kwsum
                SK = jnp.concatenate(
                    [jnp.zeros((1, dh), jnp.float32), sk], axis=0)
                R = jax.lax.dot_general(SQ, SK, (((1,), (1,)), ((), ())),
                                        preferred_element_type=jnp.float32)
                Rm = jnp.where(jc <= ir, R * scale,
                               -jnp.finfo(jnp.float32).max)
                m = jnp.max(Rm, axis=1, keepdims=True)
                e = jnp.exp(Rm - m)
                p = e / jnp.sum(e, axis=1, keepdims=True)
                Rz = jnp.where(jc < ir, p, 0.0)
                mx = jnp.max(Rz, axis=1, keepdims=True)
                cand = jnp.where(Rz >= mx, jc.astype(jnp.float32),
                                 np.float32(1e9))
                idxf = jnp.min(cand, axis=1, keepdims=True)
                mx_all[(half, sub)] = mx
                idx_rows.append(jnp.broadcast_to(idxf.astype(jnp.int32),
                                                 (nb, BSZ)))
        idx_s[...] = jnp.concatenate(idx_rows, axis=0)

        nulls = {
            (half, sub): (
                jnp.broadcast_to(nk_ref[half, sub], (BSZ, dh)),
                jnp.broadcast_to(nv_ref[half, sub], (BSZ, dh)))
            for half in range(2) for sub in range(1)
        }

        ii = jax.lax.broadcasted_iota(jnp.int32, (BSZ, BSZ), 0)
        jj = jax.lax.broadcasted_iota(jnp.int32, (BSZ, BSZ), 1)
        own_causal = jj <= ii

        def flush(groups, u):
            base = (u - gsz + 1) * BSZ
            ow0 = jnp.stack([jnp.concatenate(groups[(0, s)], axis=0)
                             for s in range(1)], axis=0)
            ow1 = jnp.stack([jnp.concatenate(groups[(1, s)], axis=0)
                             for s in range(1)], axis=0)
            out_ref[0, 0, :, base:base + gsz * BSZ, :] = ow0
            if base + gsz * BSZ == t:
                out_ref[0, 1, :, base + r:t, :] = ow1[:, 0:gsz * BSZ - r, :]
                out_ref[0, 1, :, 0:r, :] = ow1[:, gsz * BSZ - r:, :]
            else:
                out_ref[0, 1, :, base + r:base + gsz * BSZ + r, :] = ow1

        groups = {(hf, s): [] for hf in range(2) for s in range(1)}
        for u in range(nb):
            for half in range(2):
                for sub in range(1):
                    qu = bucket(0, half, sub, u)
                    ku = bucket(1, half, sub, u)
                    vu = bucket(2, half, sub, u)
                    qu16 = (qu * qscale).astype(jnp.bfloat16)
                    vu16 = vu.astype(jnp.bfloat16)

                    dots_own = jax.lax.dot_general(
                        qu16, ku.astype(jnp.bfloat16),
                        (((1,), (1,)), ((), ())),
                        preferred_element_type=jnp.float32)
                    e_own = jnp.exp2(dots_own)
                    if u == nb - 1 and half:
                        own_mask = own_causal & ~((jj == 0) & (ii >= 1))
                    else:
                        own_mask = own_causal
                    e_own = jnp.where(own_mask, e_own, 0.0)

                    if u == 0:
                        acc = jax.lax.dot_general(
                            e_own.astype(jnp.bfloat16), vu16,
                            (((1,), (0,)), ((), ())),
                            preferred_element_type=jnp.float32)
                        s = jnp.sum(e_own, axis=1, keepdims=True) \
                            + np.float32(BSZ)
                        groups[(half, sub)].append(acc / s)
                        continue

                    iu = idx_s[(half + sub) * nb + u, 0]
                    offg = pl.multiple_of(
                        jnp.maximum(iu - 1, 0) * BSZ, BSZ)
                    kgl = gather(1, half, sub, offg)
                    vgl = gather(2, half, sub, offg)
                    isnull = iu == 0
                    kg = jnp.where(isnull, nulls[(half, sub)][0], kgl)
                    vg = jnp.where(isnull, nulls[(half, sub)][1], vgl)
                    val = mx_all[(half, sub)][u:u + 1, 0:1]

                    dots_g = jax.lax.dot_general(
                        qu16, (kg * val).astype(jnp.bfloat16),
                        (((1,), (1,)), ((), ())),
                        preferred_element_type=jnp.float32)
                    e_g = jnp.exp2(dots_g)
                    if u == nb - 1 and half:
                        e_g = jnp.where(ii == 0, e_g, 0.0)

                    acc = jax.lax.dot_general(
                        e_g.astype(jnp.bfloat16),
                        (vg * val).astype(jnp.bfloat16),
                        (((1,), (0,)), ((), ())),
                        preferred_element_type=jnp.float32) \
                        + jax.lax.dot_general(
                            e_own.astype(jnp.bfloat16), vu16,
                            (((1,), (0,)), ((), ())),
                            preferred_element_type=jnp.float32)
                    s = (jnp.sum(e_g, axis=1, keepdims=True)
                         + jnp.sum(e_own, axis=1, keepdims=True))
                    groups[(half, sub)].append(acc / s)
            if len(groups[(0, 0)]) == gsz:
                flush(groups, u)
                groups = {(hf, s): [] for hf in range(2) for s in range(1)}

    return body


def kernel(q, k, v, null_keys, null_values):
    b, h, t, dh = q.shape
    nb = t // BSZ
    hh = h // 2
    nhalf = hh

    pos = np.arange(t, dtype=np.float64).reshape(nb, BSZ)
    wmat = 1.0 / (pos + 1.0)
    wsuf = np.cumsum(wmat[:, ::-1], axis=1)[:, ::-1]
    wfull = jnp.asarray(
        np.broadcast_to(wsuf.reshape(t, 1), (t, dh)), dtype=jnp.float32)
    afull = jnp.asarray(
        np.broadcast_to(wsuf[:, 0:1], (nb, dh)), dtype=jnp.float32)

    q5 = q.reshape(b, 2, hh, t, dh)
    k5 = k.reshape(b, 2, hh, t, dh)
    v5 = v.reshape(b, 2, hh, t, dh)
    nk4 = null_keys.reshape(2, hh, 1, dh)
    nv4 = null_values.reshape(2, hh, 1, dh)

    def rowmap(i):
        return (i // nhalf, 0, i % nhalf, 0, 0)

    fused = pl.pallas_call(
        _make_body(b, h, t, dh),
        grid=(b * nhalf,),
        in_specs=[
            pl.BlockSpec((t, dh), lambda i: (0, 0)),
            pl.BlockSpec((nb, dh), lambda i: (0, 0)),
            pl.BlockSpec((1, 2, 1, t, dh), rowmap),
            pl.BlockSpec((1, 2, 1, t, dh), rowmap),
            pl.BlockSpec((1, 2, 1, t, dh), rowmap),
            pl.BlockSpec((2, 1, 1, dh), lambda i: (0, i % nhalf, 0, 0)),
            pl.BlockSpec((2, 1, 1, dh), lambda i: (0, i % nhalf, 0, 0)),
        ],
        out_specs=pl.BlockSpec((1, 2, 1, t, dh), rowmap),
        out_shape=jax.ShapeDtypeStruct((b, 2, hh, t, dh), jnp.float32),
        scratch_shapes=[
            pltpu.VMEM((2 * nb, BSZ), jnp.int32),
            pltpu.VMEM((t, dh), jnp.float32),
            pltpu.VMEM((t, dh), jnp.float32),
            pltpu.VMEM((t, dh), jnp.float32),
        ],
        compiler_params=pltpu.CompilerParams(
            dimension_semantics=("arbitrary",)),
    )
    out5 = fused(wfull, afull, q5, k5, v5, nk4, nv4)
    return out5.reshape(b, h, t, dh)
